# Initial kernel scaffold; baseline (speedup 1.0000x reference)
import jax
import jax.numpy as jnp
from jax.experimental import pallas as pl


def kernel(mask, weight):
    raise NotImplementedError("write your pallas kernel here")


# speedup vs baseline: 1.8062x; 1.8062x over previous
import functools

import jax
import jax.numpy as jnp
from jax import lax
from jax.experimental import pallas as pl
from jax.experimental.pallas import tpu as pltpu
from jax.experimental.pallas import tpu_sc as plsc

VOCAB = 1000000
EMB = 64
B = 16384
L = 50
N = B * L

_info = plsc.get_sparse_core_info()
NC, NS = _info.num_cores, _info.num_subcores
NW = NC * NS
PER_W = N // NW
CHUNK = 512
N_CHUNKS = PER_W // CHUNK

_mesh = plsc.VectorSubcoreMesh(core_axis_name="c", subcore_axis_name="s")


@functools.partial(
    pl.kernel,
    mesh=_mesh,
    out_type=jax.ShapeDtypeStruct((N, EMB), jnp.float32),
    scratch_types=[
        pltpu.VMEM((CHUNK,), jnp.int32),
        pltpu.VMEM((CHUNK, EMB), jnp.float32),
        pltpu.SemaphoreType.DMA,
    ],
    compiler_params=pltpu.CompilerParams(use_tc_tiling_on_sc=False),
)
def _gather_kernel(idx_hbm, table_hbm, out_hbm, idx_v, rows_v, sem):
    wid = lax.axis_index("s") * NC + lax.axis_index("c")
    base = wid * PER_W

    @pl.loop(0, N_CHUNKS)
    def _chunk(g):
        start = base + g * CHUNK
        pltpu.sync_copy(idx_hbm.at[pl.ds(start, CHUNK)], idx_v)
        pltpu.async_copy(table_hbm.at[idx_v], rows_v, sem).wait()
        pltpu.sync_copy(rows_v, out_hbm.at[pl.ds(start, CHUNK)])


def kernel(mask, weight):
    flat = mask.reshape(N)
    out = _gather_kernel(flat, weight)
    return out.reshape(B, L, EMB)


# speedup vs baseline: 1.8738x; 1.0374x over previous
import functools

import jax
import jax.numpy as jnp
from jax import lax
from jax.experimental import pallas as pl
from jax.experimental.pallas import tpu as pltpu
from jax.experimental.pallas import tpu_sc as plsc

VOCAB = 1000000
EMB = 64
B = 16384
L = 50
N = B * L

_info = plsc.get_sparse_core_info()
NC, NS = _info.num_cores, _info.num_subcores
NW = NC * NS
PER_W = N // NW
CHUNK = 640
N_CHUNKS = PER_W // CHUNK

_mesh = plsc.VectorSubcoreMesh(core_axis_name="c", subcore_axis_name="s")


@functools.partial(
    pl.kernel,
    mesh=_mesh,
    out_type=jax.ShapeDtypeStruct((N, EMB), jnp.float32),
    scratch_types=[
        pltpu.VMEM((PER_W,), jnp.int32),
        pltpu.VMEM((2, CHUNK, EMB), jnp.float32),
        pltpu.SemaphoreType.DMA,
        pltpu.SemaphoreType.DMA,
        pltpu.SemaphoreType.DMA,
        pltpu.SemaphoreType.DMA,
    ],
    compiler_params=pltpu.CompilerParams(use_tc_tiling_on_sc=False),
)
def _gather_kernel(idx_hbm, table_hbm, out_hbm, idx_v, rows_v, sg0, sg1, sw0, sw1):
    wid = lax.axis_index("s") * NC + lax.axis_index("c")
    base = wid * PER_W
    sg = [sg0, sg1]
    sw = [sw0, sw1]

    pltpu.sync_copy(idx_hbm.at[pl.ds(base, PER_W)], idx_v)

    def gather_start(g, b):
        pltpu.async_copy(
            table_hbm.at[idx_v.at[pl.ds(g * CHUNK, CHUNK)]], rows_v.at[b], sg[b]
        )

    def gather_wait(g, b):
        pltpu.make_async_copy(
            table_hbm.at[idx_v.at[pl.ds(g * CHUNK, CHUNK)]], rows_v.at[b], sg[b]
        ).wait()

    def wb_start(g, b):
        pltpu.async_copy(rows_v.at[b], out_hbm.at[pl.ds(base + g * CHUNK, CHUNK)], sw[b])

    def wb_wait(g, b):
        pltpu.make_async_copy(
            rows_v.at[b], out_hbm.at[pl.ds(base + g * CHUNK, CHUNK)], sw[b]
        ).wait()

    gather_start(0, 0)

    @pl.loop(0, N_CHUNKS, step=2)
    def _body(g):
        gather_wait(g, 0)

        @pl.when(g >= 2)
        def _():
            wb_wait(g - 1, 1)

        gather_start(g + 1, 1)
        wb_start(g, 0)

        gather_wait(g + 1, 1)
        wb_wait(g, 0)

        @pl.when(g + 2 < N_CHUNKS)
        def _():
            gather_start(g + 2, 0)

        wb_start(g + 1, 1)

    wb_wait(N_CHUNKS - 1, 1)


def kernel(mask, weight):
    flat = mask.reshape(N)
    out = _gather_kernel(flat, weight)
    return out.reshape(B, L, EMB)


# speedup vs baseline: 2.2884x; 1.2213x over previous
import functools

import jax
import jax.numpy as jnp
from jax import lax
from jax.experimental import pallas as pl
from jax.experimental.pallas import tpu as pltpu
from jax.experimental.pallas import tpu_sc as plsc

VOCAB = 1000000
EMB = 64
B = 16384
L = 50
N = B * L
EP = 128
LP = 56

_info = plsc.get_sparse_core_info()
NC, NS = _info.num_cores, _info.num_subcores
NW = NC * NS
PER_W = N // NW
B_PER_W = B // NW
CB = 4
CHUNK = CB * L
N_CHUNKS = B_PER_W // CB

_mesh = plsc.VectorSubcoreMesh(core_axis_name="c", subcore_axis_name="s")


@functools.partial(
    pl.kernel,
    mesh=_mesh,
    out_type=jax.ShapeDtypeStruct((B, LP, EP), jnp.float32),
    scratch_types=[
        pltpu.VMEM((PER_W,), jnp.int32),
        pltpu.VMEM((2, CHUNK, EP), jnp.float32),
        pltpu.SemaphoreType.DMA,
        pltpu.SemaphoreType.DMA,
        pltpu.SemaphoreType.DMA,
        pltpu.SemaphoreType.DMA,
    ],
    compiler_params=pltpu.CompilerParams(use_tc_tiling_on_sc=False),
)
def _gather_kernel(idx_hbm, table_hbm, out_hbm, idx_v, rows_v, sg0, sg1, sw0, sw1):
    wid = lax.axis_index("s") * NC + lax.axis_index("c")
    base = wid * PER_W
    b0 = wid * B_PER_W
    sg = [sg0, sg1]
    sw = [sw0, sw1]

    pltpu.sync_copy(idx_hbm.at[pl.ds(base, PER_W)], idx_v)

    def gather_start(g, b):
        pltpu.async_copy(
            table_hbm.at[idx_v.at[pl.ds(g * CHUNK, CHUNK)]], rows_v.at[b], sg[b]
        )

    def gather_wait(g, b):
        pltpu.make_async_copy(
            table_hbm.at[idx_v.at[pl.ds(g * CHUNK, CHUNK)]], rows_v.at[b], sg[b]
        ).wait()

    def wb_start(g, b):
        for k in range(CB):
            pltpu.async_copy(
                rows_v.at[b].at[pl.ds(k * L, L)],
                out_hbm.at[b0 + g * CB + k, pl.ds(0, L), :],
                sw[b],
            )

    def wb_wait(g, b):
        for k in range(CB):
            pltpu.make_async_copy(
                rows_v.at[b].at[pl.ds(k * L, L)],
                out_hbm.at[b0 + g * CB + k, pl.ds(0, L), :],
                sw[b],
            ).wait()

    gather_start(0, 0)

    @pl.loop(0, N_CHUNKS, step=2)
    def _body(g):
        gather_wait(g, 0)

        @pl.when(g >= 2)
        def _():
            wb_wait(g - 1, 1)

        gather_start(g + 1, 1)
        wb_start(g, 0)

        gather_wait(g + 1, 1)
        wb_wait(g, 0)

        @pl.when(g + 2 < N_CHUNKS)
        def _():
            gather_start(g + 2, 0)

        wb_start(g + 1, 1)

    wb_wait(N_CHUNKS - 1, 1)


def kernel(mask, weight):
    wpad = jnp.pad(weight, ((0, 0), (0, EP - EMB)))
    flat = mask.reshape(N)
    out = _gather_kernel(flat, wpad)
    return out[:, :L, :EMB]


# speedup vs baseline: 2.5161x; 1.0995x over previous
import functools

import jax
import jax.numpy as jnp
from jax import lax
from jax.experimental import pallas as pl
from jax.experimental.pallas import tpu as pltpu
from jax.experimental.pallas import tpu_sc as plsc

VOCAB = 1000000
EMB = 64
B = 16384
L = 50
N = B * L
EP = 128
LP = 56

_info = plsc.get_sparse_core_info()
NC, NS = _info.num_cores, _info.num_subcores
NW = NC * NS
PER_W = N // NW
B_PER_W = B // NW
CB = 8
CHUNK = CB * L
N_CHUNKS = B_PER_W // CB

_mesh = plsc.VectorSubcoreMesh(core_axis_name="c", subcore_axis_name="s")


@functools.partial(
    pl.kernel,
    mesh=_mesh,
    out_type=jax.ShapeDtypeStruct((B, LP, EP), jnp.float32),
    scratch_types=[
        pltpu.VMEM((PER_W,), jnp.int32),
        pltpu.VMEM((2, CHUNK, EMB), jnp.float32),
        pltpu.SemaphoreType.DMA,
        pltpu.SemaphoreType.DMA,
        pltpu.SemaphoreType.DMA,
        pltpu.SemaphoreType.DMA,
    ],
    compiler_params=pltpu.CompilerParams(use_tc_tiling_on_sc=False),
)
def _gather_kernel(idx_hbm, table_hbm, out_hbm, idx_v, rows_v, sg0, sg1, sw0, sw1):
    wid = lax.axis_index("s") * NC + lax.axis_index("c")
    base = wid * PER_W
    b0 = wid * B_PER_W
    sg = [sg0, sg1]
    sw = [sw0, sw1]

    pltpu.sync_copy(idx_hbm.at[pl.ds(base, PER_W)], idx_v)

    def gather_start(g, b):
        pltpu.async_copy(
            table_hbm.at[idx_v.at[pl.ds(g * CHUNK, CHUNK)]], rows_v.at[b], sg[b]
        )

    def gather_wait(g, b):
        pltpu.make_async_copy(
            table_hbm.at[idx_v.at[pl.ds(g * CHUNK, CHUNK)]], rows_v.at[b], sg[b]
        ).wait()

    def wb_start(g, b):
        for k in range(CB):
            pltpu.async_copy(
                rows_v.at[b].at[pl.ds(k * L, L)],
                out_hbm.at[b0 + g * CB + k, pl.ds(0, L), pl.ds(0, EMB)],
                sw[b],
            )

    def wb_wait(g, b):
        for k in range(CB):
            pltpu.make_async_copy(
                rows_v.at[b].at[pl.ds(k * L, L)],
                out_hbm.at[b0 + g * CB + k, pl.ds(0, L), pl.ds(0, EMB)],
                sw[b],
            ).wait()

    gather_start(0, 0)

    @pl.loop(0, N_CHUNKS, step=2)
    def _body(g):
        gather_wait(g, 0)

        @pl.when(g >= 2)
        def _():
            wb_wait(g - 1, 1)

        gather_start(g + 1, 1)
        wb_start(g, 0)

        gather_wait(g + 1, 1)
        wb_wait(g, 0)

        @pl.when(g + 2 < N_CHUNKS)
        def _():
            gather_start(g + 2, 0)

        wb_start(g + 1, 1)

    wb_wait(N_CHUNKS - 1, 1)


def kernel(mask, weight):
    flat = mask.reshape(N)
    out = _gather_kernel(flat, weight)
    return out[:, :L, :EMB]


# speedup vs baseline: 2.5164x; 1.0001x over previous
import functools

import jax
import jax.numpy as jnp
from jax import lax
from jax.experimental import pallas as pl
from jax.experimental.pallas import tpu as pltpu
from jax.experimental.pallas import tpu_sc as plsc

VOCAB = 1000000
EMB = 64
B = 16384
L = 50
N = B * L
EP = 128
LP = 56

_info = plsc.get_sparse_core_info()
NC, NS = _info.num_cores, _info.num_subcores
NW = NC * NS
PER_W = N // NW
B_PER_W = B // NW
CB = 8
CHUNK = CB * L
N_CHUNKS = B_PER_W // CB

_mesh = plsc.VectorSubcoreMesh(core_axis_name="c", subcore_axis_name="s")


@functools.partial(
    pl.kernel,
    mesh=_mesh,
    out_type=jax.ShapeDtypeStruct((B, LP, EP), jnp.float32),
    scratch_types=[
        pltpu.VMEM((PER_W,), jnp.int32),
        pltpu.VMEM((2, CHUNK, EMB), jnp.float32),
        pltpu.SemaphoreType.DMA,
        pltpu.SemaphoreType.DMA,
        pltpu.SemaphoreType.DMA,
        pltpu.SemaphoreType.DMA,
    ],
    compiler_params=pltpu.CompilerParams(
        use_tc_tiling_on_sc=False,
        skip_device_barrier=True,
        disable_bounds_checks=True,
        disable_semaphore_checks=True,
    ),
)
def _gather_kernel(idx_hbm, table_hbm, out_hbm, idx_v, rows_v, sg0, sg1, sw0, sw1):
    wid = lax.axis_index("s") * NC + lax.axis_index("c")
    base = wid * PER_W
    b0 = wid * B_PER_W
    sg = [sg0, sg1]
    sw = [sw0, sw1]

    pltpu.sync_copy(idx_hbm.at[pl.ds(base, PER_W)], idx_v)

    def gather_start(g, b):
        pltpu.async_copy(
            table_hbm.at[idx_v.at[pl.ds(g * CHUNK, CHUNK)]], rows_v.at[b], sg[b]
        )

    def gather_wait(g, b):
        pltpu.make_async_copy(
            table_hbm.at[idx_v.at[pl.ds(g * CHUNK, CHUNK)]], rows_v.at[b], sg[b]
        ).wait()

    def wb_start(g, b):
        for k in range(CB):
            pltpu.async_copy(
                rows_v.at[b].at[pl.ds(k * L, L)],
                out_hbm.at[b0 + g * CB + k, pl.ds(0, L), pl.ds(0, EMB)],
                sw[b],
            )

    def wb_wait(g, b):
        for k in range(CB):
            pltpu.make_async_copy(
                rows_v.at[b].at[pl.ds(k * L, L)],
                out_hbm.at[b0 + g * CB + k, pl.ds(0, L), pl.ds(0, EMB)],
                sw[b],
            ).wait()

    gather_start(0, 0)

    @pl.loop(0, N_CHUNKS, step=2)
    def _body(g):
        gather_wait(g, 0)

        @pl.when(g >= 2)
        def _():
            wb_wait(g - 1, 1)

        gather_start(g + 1, 1)
        wb_start(g, 0)

        gather_wait(g + 1, 1)
        wb_wait(g, 0)

        @pl.when(g + 2 < N_CHUNKS)
        def _():
            gather_start(g + 2, 0)

        wb_start(g + 1, 1)

    wb_wait(N_CHUNKS - 1, 1)


def kernel(mask, weight):
    flat = mask.reshape(N)
    out = _gather_kernel(flat, weight)
    return out[:, :L, :EMB]


# speedup vs baseline: 2.5312x; 1.0059x over previous
import functools

import jax
import jax.numpy as jnp
from jax import lax
from jax.experimental import pallas as pl
from jax.experimental.pallas import tpu as pltpu
from jax.experimental.pallas import tpu_sc as plsc

VOCAB = 1000000
EMB = 64
B = 16384
L = 50
N = B * L
EP = 128
LP = 56

_info = plsc.get_sparse_core_info()
NC, NS = _info.num_cores, _info.num_subcores
NW = NC * NS
PER_W = N // NW
B_PER_W = B // NW
CB = 4
CHUNK = CB * L
N_CHUNKS = B_PER_W // CB
NBUF = 4

_mesh = plsc.VectorSubcoreMesh(core_axis_name="c", subcore_axis_name="s")


@functools.partial(
    pl.kernel,
    mesh=_mesh,
    out_type=jax.ShapeDtypeStruct((B, LP, EP), jnp.float32),
    scratch_types=[
        pltpu.VMEM((PER_W,), jnp.int32),
        pltpu.VMEM((NBUF, CHUNK, EMB), jnp.float32),
    ]
    + [pltpu.SemaphoreType.DMA] * (2 * NBUF),
    compiler_params=pltpu.CompilerParams(
        use_tc_tiling_on_sc=False,
        skip_device_barrier=True,
        disable_bounds_checks=True,
        disable_semaphore_checks=True,
    ),
)
def _gather_kernel(idx_hbm, table_hbm, out_hbm, idx_v, rows_v, *sems):
    wid = lax.axis_index("s") * NC + lax.axis_index("c")
    base = wid * PER_W
    b0 = wid * B_PER_W
    sg = sems[:NBUF]
    sw = sems[NBUF:]

    pltpu.sync_copy(idx_hbm.at[pl.ds(base, PER_W)], idx_v)

    def gather_start(g, b):
        pltpu.async_copy(
            table_hbm.at[idx_v.at[pl.ds(g * CHUNK, CHUNK)]], rows_v.at[b], sg[b]
        )

    def gather_wait(g, b):
        pltpu.make_async_copy(
            table_hbm.at[idx_v.at[pl.ds(g * CHUNK, CHUNK)]], rows_v.at[b], sg[b]
        ).wait()

    def wb_start(g, b):
        for k in range(CB):
            pltpu.async_copy(
                rows_v.at[b].at[pl.ds(k * L, L)],
                out_hbm.at[b0 + g * CB + k, pl.ds(0, L), pl.ds(0, EMB)],
                sw[b],
            )

    def wb_wait(g, b):
        for k in range(CB):
            pltpu.make_async_copy(
                rows_v.at[b].at[pl.ds(k * L, L)],
                out_hbm.at[b0 + g * CB + k, pl.ds(0, L), pl.ds(0, EMB)],
                sw[b],
            ).wait()

    gather_start(0, 0)
    gather_start(1, 1)

    @pl.loop(0, N_CHUNKS, step=NBUF)
    def _body(g):
        for j in range(NBUF):
            c = g + j
            b = j

            gather_wait(c, b)
            wb_start(c, b)

            @pl.when(c + 2 < N_CHUNKS)
            def _(c=c, b=b):
                @pl.when(c >= 2)
                def _():
                    wb_wait(c - 2, (b + 2) % NBUF)

                gather_start(c + 2, (b + 2) % NBUF)

    for c in (N_CHUNKS - 2, N_CHUNKS - 1):
        wb_wait(c, c % NBUF)


def kernel(mask, weight):
    flat = mask.reshape(N)
    out = _gather_kernel(flat, weight)
    return out[:, :L, :EMB]


# speedup vs baseline: 2.5455x; 1.0057x over previous
import functools

import jax
import jax.numpy as jnp
from jax import lax
from jax.experimental import pallas as pl
from jax.experimental.pallas import tpu as pltpu
from jax.experimental.pallas import tpu_sc as plsc

VOCAB = 1000000
EMB = 64
B = 16384
L = 50
N = B * L
EP = 128
LP = 56

_info = plsc.get_sparse_core_info()
NC, NS = _info.num_cores, _info.num_subcores
NW = NC * NS
PER_W = N // NW
B_PER_W = B // NW
CB = 4
CHUNK = CB * L
N_CHUNKS = B_PER_W // CB
NBUF = 4

_mesh = plsc.VectorSubcoreMesh(core_axis_name="c", subcore_axis_name="s")


@functools.partial(
    pl.kernel,
    mesh=_mesh,
    out_type=jax.ShapeDtypeStruct((B, LP, EP), jnp.float32),
    scratch_types=[
        pltpu.VMEM((PER_W,), jnp.int32),
        pltpu.VMEM((NBUF, CHUNK, EMB), jnp.float32),
    ]
    + [pltpu.SemaphoreType.DMA] * (2 * NBUF),
    compiler_params=pltpu.CompilerParams(use_tc_tiling_on_sc=False),
)
def _gather_kernel(idx_hbm, table_hbm, out_hbm, idx_v, rows_v, *sems):
    wid = lax.axis_index("s") * NC + lax.axis_index("c")
    base = wid * PER_W
    b0 = wid * B_PER_W
    sg = sems[:NBUF]
    sw = sems[NBUF:]

    pltpu.sync_copy(idx_hbm.at[pl.ds(base, PER_W)], idx_v)

    def gather_start(g, b):
        pltpu.async_copy(
            table_hbm.at[idx_v.at[pl.ds(g * CHUNK, CHUNK)]], rows_v.at[b], sg[b]
        )

    def gather_wait(g, b):
        pltpu.make_async_copy(
            table_hbm.at[idx_v.at[pl.ds(g * CHUNK, CHUNK)]], rows_v.at[b], sg[b]
        ).wait()

    def wb_start(g, b):
        for k in range(CB):
            pltpu.async_copy(
                rows_v.at[b].at[pl.ds(k * L, L)],
                out_hbm.at[b0 + g * CB + k, pl.ds(0, L), pl.ds(0, EMB)],
                sw[b],
            )

    def wb_wait(g, b):
        for k in range(CB):
            pltpu.make_async_copy(
                rows_v.at[b].at[pl.ds(k * L, L)],
                out_hbm.at[b0 + g * CB + k, pl.ds(0, L), pl.ds(0, EMB)],
                sw[b],
            ).wait()

    gather_start(0, 0)
    gather_start(1, 1)

    @pl.loop(0, N_CHUNKS, step=NBUF)
    def _body(g):
        for j in range(NBUF):
            c = g + j
            b = j

            gather_wait(c, b)
            wb_start(c, b)

            @pl.when(c + 2 < N_CHUNKS)
            def _(c=c, b=b):
                @pl.when(c >= 2)
                def _():
                    wb_wait(c - 2, (b + 2) % NBUF)

                gather_start(c + 2, (b + 2) % NBUF)

    for c in (N_CHUNKS - 2, N_CHUNKS - 1):
        wb_wait(c, c % NBUF)


def kernel(mask, weight):
    flat = mask.reshape(N)
    out = _gather_kernel(flat, weight)
    return out[:, :L, :EMB]
